# Initial kernel scaffold; baseline (speedup 1.0000x reference)
import jax
import jax.numpy as jnp
from jax.experimental import pallas as pl


def kernel(h, edge_weight, edge_feat, params, edge_index):
    raise NotImplementedError("write your pallas kernel here")


# speedup vs baseline: 2.1071x; 2.1071x over previous
import functools

import jax
import jax.numpy as jnp
from jax import lax
from jax.experimental import pallas as pl
from jax.experimental.pallas import tpu as pltpu
from jax.experimental.pallas import tpu_sc as plsc

N = 10000
E = 160000
D = 256
HD = 128
NSUB = 16
B = 128
ES_SUB = 10240
EPAD = ES_SUB * NSUB
EPADX = EPAD + B
NBATCH = ES_SUB // B
ES32 = EPAD // 32
NBATCH32 = ES32 // B
NPAD = 10112
ZROWS = 632
ZCH = 128
ZSIZES = (128, 128, 128, 128, 120)
EPS = 1e-5
BN = 1000
BE = 2000


def _sc_mesh():
    return plsc.VectorSubcoreMesh(core_axis_name="c", subcore_axis_name="s")


@functools.partial(
    pl.kernel,
    out_type=jax.ShapeDtypeStruct((2, N, HD), jnp.float32),
    mesh=_sc_mesh(),
    scratch_types=[
        pltpu.VMEM((2 * B,), jnp.int32),
        pltpu.VMEM((NBATCH, B), jnp.int32),
        pltpu.VMEM((2 * B * 16,), jnp.float32),
        pltpu.VMEM((B, HD), jnp.float32),
        pltpu.VMEM((B, HD), jnp.float32),
        pltpu.VMEM_SHARED((NPAD, HD), jnp.float32),
        pltpu.SemaphoreType.DMA,
        pltpu.SemaphoreType.DMA,
        pltpu.SemaphoreType.DMA,
        pltpu.SemaphoreType.DMA,
    ],
)
def _sc_agg(hh_flat, srcp, dst3, wp16, out,
            is_pair, dst_all, wv, rows0, rows1, aggsh,
            gs0, gs1, ss0, ss1):
    cid = lax.axis_index("c")
    sid = lax.axis_index("s")
    rows = (rows0, rows1)
    gsem = (gs0, gs1)
    ssem = (ss0, ss1)

    def zrow(r, carry):
        for ch in range(HD // 16):
            rows0[r, pl.ds(ch * 16, 16)] = jnp.zeros((16,), jnp.float32)
        return carry
    lax.fori_loop(0, ZCH, zrow, 0)
    off0 = 0
    for sz in ZSIZES:
        pltpu.sync_copy(rows0.at[pl.ds(0, sz)],
                        aggsh.at[pl.ds(sid * ZROWS + off0, sz)])
        off0 += sz
    pltpu.sync_copy(dst3.at[sid], dst_all)
    plsc.subcore_barrier()

    base = sid * ES_SUB
    roff = cid * N

    def pair(i, carry):
        off = base + i * (2 * B)
        pltpu.sync_copy(srcp.at[pl.ds(off, 2 * B)], is_pair)
        pltpu.sync_copy(wp16.at[pl.ds(off * 16, 2 * B * 16)], wv)
        for ch in range(2 * B // 16):
            is_pair[pl.ds(ch * 16, 16)] = is_pair[pl.ds(ch * 16, 16)] + roff
        for j in range(2):
            pltpu.async_copy(hh_flat.at[is_pair.at[pl.ds(j * B, B)]],
                             rows[j], gsem[j])
        for j in range(2):
            g = i * 2 + j
            pltpu.make_async_copy(hh_flat.at[is_pair.at[pl.ds(j * B, B)]],
                                  rows[j], gsem[j]).wait()

            def edge(e, c2):
                wbc = wv[pl.ds((j * B + e) * 16, 16)]
                for ch in range(HD // 16):
                    rows[j][e, pl.ds(ch * 16, 16)] = (
                        rows[j][e, pl.ds(ch * 16, 16)] * wbc)
                return c2
            lax.fori_loop(0, B, edge, 0)
            pltpu.async_copy(rows[j], aggsh.at[dst_all.at[g]], ssem[j],
                             add=True)
        for j in range(2):
            g = i * 2 + j
            pltpu.make_async_copy(rows[j], aggsh.at[dst_all.at[g]],
                                  ssem[j]).wait()
        return carry
    lax.fori_loop(0, NBATCH // 2, pair, 0)
    plsc.subcore_barrier()

    @pl.when(sid < 10)
    def _():
        pltpu.sync_copy(aggsh.at[pl.ds(sid * 1000, 1000)],
                        out.at[cid, pl.ds(sid * 1000, 1000)])


@functools.partial(
    pl.kernel,
    out_type=jax.ShapeDtypeStruct((2, N, HD), jnp.float32),
    mesh=_sc_mesh(),
    scratch_types=[
        pltpu.VMEM((B,), jnp.int32),
        pltpu.VMEM((B, HD), jnp.float32),
        pltpu.VMEM((ZCH, HD), jnp.float32),
        pltpu.VMEM_SHARED((NPAD, HD), jnp.float32),
    ],
)
def _sc_deg(dstp, out, idx_d, onesb, zbuf, degsh):
    cid = lax.axis_index("c")
    sid = lax.axis_index("s")

    def frow(r, carry):
        for ch in range(HD // 16):
            zbuf[r, pl.ds(ch * 16, 16)] = jnp.zeros((16,), jnp.float32)
            onesb[r, pl.ds(ch * 16, 16)] = jnp.ones((16,), jnp.float32)
        return carry
    lax.fori_loop(0, ZCH, frow, 0)
    off0 = 0
    for sz in ZSIZES:
        pltpu.sync_copy(zbuf.at[pl.ds(0, sz)],
                        degsh.at[pl.ds(sid * ZROWS + off0, sz)])
        off0 += sz
    plsc.subcore_barrier()

    base = (cid * NSUB + sid) * ES32

    def batch(b, carry):
        pltpu.sync_copy(dstp.at[pl.ds(base + b * B, B)], idx_d)
        pltpu.sync_copy(onesb, degsh.at[idx_d], add=True)
        return carry
    lax.fori_loop(0, NBATCH32, batch, 0)
    plsc.subcore_barrier()

    @pl.when(sid < 10)
    def _():
        pltpu.sync_copy(degsh.at[pl.ds(sid * 1000, 1000)],
                        out.at[cid, pl.ds(sid * 1000, 1000)])


@functools.partial(
    pl.kernel,
    out_type=jax.ShapeDtypeStruct((2, EPAD, HD), jnp.float32),
    mesh=_sc_mesh(),
    scratch_types=[
        pltpu.VMEM((NBATCH, B), jnp.int32),
        pltpu.VMEM((NBATCH, B), jnp.int32),
        pltpu.VMEM((B, HD), jnp.float32),
        pltpu.VMEM((B, HD), jnp.float32),
        pltpu.VMEM((B, HD), jnp.float32),
        pltpu.VMEM((B, HD), jnp.float32),
        pltpu.SemaphoreType.DMA,
        pltpu.SemaphoreType.DMA,
        pltpu.SemaphoreType.DMA,
        pltpu.SemaphoreType.DMA,
        pltpu.SemaphoreType.DMA,
        pltpu.SemaphoreType.DMA,
    ],
)
def _sc_edge_gather(a_flat, b_flat, src3, dst3, out,
                    src_all, dst_all, ra0, ra1, rb0, rb1,
                    ga0, ga1, gb0, gb1, os0, os1):
    cid = lax.axis_index("c")
    sid = lax.axis_index("s")
    ra = (ra0, ra1)
    rb = (rb0, rb1)
    gsa = (ga0, ga1)
    gsb = (gb0, gb1)
    osem = (os0, os1)
    base = sid * ES_SUB
    roff = cid * N

    pltpu.sync_copy(src3.at[sid], src_all)
    pltpu.sync_copy(dst3.at[sid], dst_all)

    def adj(r, carry):
        for ch in range(B // 16):
            src_all[r, pl.ds(ch * 16, 16)] = (
                src_all[r, pl.ds(ch * 16, 16)] + roff)
            dst_all[r, pl.ds(ch * 16, 16)] = (
                dst_all[r, pl.ds(ch * 16, 16)] + roff)
        return carry
    lax.fori_loop(0, NBATCH, adj, 0)

    def pair(i, carry):
        for j in range(2):
            g = i * 2 + j
            pltpu.async_copy(a_flat.at[src_all.at[g]], ra[j], gsa[j])
            pltpu.async_copy(b_flat.at[dst_all.at[g]], rb[j], gsb[j])
        for j in range(2):
            g = i * 2 + j
            pltpu.make_async_copy(a_flat.at[src_all.at[g]], ra[j],
                                  gsa[j]).wait()
            pltpu.make_async_copy(b_flat.at[dst_all.at[g]], rb[j],
                                  gsb[j]).wait()

            def edge(e, c2):
                for ch in range(HD // 16):
                    ra[j][e, pl.ds(ch * 16, 16)] = (
                        ra[j][e, pl.ds(ch * 16, 16)]
                        + rb[j][e, pl.ds(ch * 16, 16)])
                return c2
            lax.fori_loop(0, B, edge, 0)
            pltpu.async_copy(ra[j], out.at[cid, pl.ds(base + g * B, B)],
                             osem[j])
        for j in range(2):
            pltpu.make_async_copy(ra[j], out.at[cid, pl.ds(0, B)],
                                  osem[j]).wait()
        return carry
    lax.fori_loop(0, NBATCH // 2, pair, 0)


def _ln_rows(z, g, b):
    mu = jnp.mean(z, axis=-1, keepdims=True)
    zc = z - mu
    var = jnp.mean(zc * zc, axis=-1, keepdims=True)
    return zc * lax.rsqrt(var + EPS) * g + b


def _proj_body(h_ref, w0, c0, g0, b0, w1, c1, g1, b1, out_ref):
    x0 = h_ref[:, :HD]
    x1 = h_ref[:, HD:]
    z0 = jnp.dot(x0, w0[...], preferred_element_type=jnp.float32) + c0[...]
    z0 = jnp.maximum(_ln_rows(z0, g0[...], b0[...]), 0.0)
    z1 = jnp.dot(x1, w1[...], preferred_element_type=jnp.float32) + c1[...]
    z1 = jnp.maximum(_ln_rows(z1, g1[...], b1[...]), 0.0)
    out_ref[0] = z0
    out_ref[1] = z1


def _layer_body(hA, hB, aA, aB, dA, dB, wself, bself, wneigh, g, beta, out_ref):
    hh = jnp.concatenate([hA[0], hB[0]], axis=1)
    deg = dA[0][:, 0:1] + dB[0][:, 0:1]
    recip = 1.0 / jnp.maximum(deg, 1.0)
    mean = jnp.concatenate([aA[0], aB[0]], axis=1) * recip
    rst = (jnp.dot(hh, wself[...], preferred_element_type=jnp.float32)
           + bself[...]
           + jnp.dot(mean, wneigh[...], preferred_element_type=jnp.float32))
    z = _ln_rows(jnp.maximum(rst, 0.0), g[...], beta[...])
    out_ref[0] = z[:, :HD]
    out_ref[1] = z[:, HD:]


def _ab_body(hA, hB, w1a, w1b, a_ref, b_ref):
    hh = jnp.concatenate([hA[0], hB[0]], axis=1)
    a = jnp.dot(hh, w1a[...], preferred_element_type=jnp.float32)
    b = jnp.dot(hh, w1b[...], preferred_element_type=jnp.float32)
    a_ref[0] = a[:, :HD]
    a_ref[1] = a[:, HD:]
    b_ref[0] = b[:, :HD]
    b_ref[1] = b[:, HD:]


def _edge_body(xa, xb, ef, b1, g, beta, w2a, w2b, b2, out_ref):
    x = jnp.concatenate([xa[0], xb[0]], axis=1) + b1[...]
    x = jnp.maximum(_ln_rows(x, g[...], beta[...]), 0.0)
    s = (jnp.dot(x, w2a[...], preferred_element_type=jnp.float32)
         + jnp.dot(ef[...], w2b[...], preferred_element_type=jnp.float32)
         + b2[...])
    out_ref[...] = s


def _const2(shape):
    return pl.BlockSpec(shape, lambda i: tuple(0 for _ in shape))


def _run_proj(h, p):
    return pl.pallas_call(
        _proj_body,
        grid=(N // BN,),
        in_specs=[
            pl.BlockSpec((BN, D), lambda i: (i, 0)),
            _const2((HD, HD)), _const2((1, HD)), _const2((1, HD)), _const2((1, HD)),
            _const2((HD, HD)), _const2((1, HD)), _const2((1, HD)), _const2((1, HD)),
        ],
        out_specs=pl.BlockSpec((2, BN, HD), lambda i: (0, i, 0)),
        out_shape=jax.ShapeDtypeStruct((2, N, HD), jnp.float32),
    )(h,
      p['Wp0'].T, p['cp0'].reshape(1, HD), p['gp0'].reshape(1, HD), p['betap0'].reshape(1, HD),
      p['Wp1'].T, p['cp1'].reshape(1, HD), p['gp1'].reshape(1, HD), p['betap1'].reshape(1, HD))


def _run_layer(hh_s, agg, deg, p, l):
    return pl.pallas_call(
        _layer_body,
        grid=(N // BN,),
        in_specs=[
            pl.BlockSpec((1, BN, HD), lambda i: (0, i, 0)),
            pl.BlockSpec((1, BN, HD), lambda i: (1, i, 0)),
            pl.BlockSpec((1, BN, HD), lambda i: (0, i, 0)),
            pl.BlockSpec((1, BN, HD), lambda i: (1, i, 0)),
            pl.BlockSpec((1, BN, HD), lambda i: (0, i, 0)),
            pl.BlockSpec((1, BN, HD), lambda i: (1, i, 0)),
            _const2((D, D)), _const2((1, D)), _const2((D, D)),
            _const2((1, D)), _const2((1, D)),
        ],
        out_specs=pl.BlockSpec((2, BN, HD), lambda i: (0, i, 0)),
        out_shape=jax.ShapeDtypeStruct((2, N, HD), jnp.float32),
    )(hh_s, hh_s, agg, agg, deg, deg,
      p[f'Wself{l}'].T, p[f'bself{l}'].reshape(1, D), p[f'Wneigh{l}'].T,
      p[f'g{l}'].reshape(1, D), p[f'beta{l}'].reshape(1, D))


def _run_ab(hh_s, p):
    w1 = p['W1']
    return pl.pallas_call(
        _ab_body,
        grid=(N // BN,),
        in_specs=[
            pl.BlockSpec((1, BN, HD), lambda i: (0, i, 0)),
            pl.BlockSpec((1, BN, HD), lambda i: (1, i, 0)),
            _const2((D, D)), _const2((D, D)),
        ],
        out_specs=[
            pl.BlockSpec((2, BN, HD), lambda i: (0, i, 0)),
            pl.BlockSpec((2, BN, HD), lambda i: (0, i, 0)),
        ],
        out_shape=[
            jax.ShapeDtypeStruct((2, N, HD), jnp.float32),
            jax.ShapeDtypeStruct((2, N, HD), jnp.float32),
        ],
    )(hh_s, hh_s, w1[:, :D].T, w1[:, D:].T)


def _run_edge_mlp(xs, edge_feat, p):
    w2 = p['W2']
    return pl.pallas_call(
        _edge_body,
        grid=(E // BE,),
        in_specs=[
            pl.BlockSpec((1, BE, HD), lambda i: (0, i, 0)),
            pl.BlockSpec((1, BE, HD), lambda i: (1, i, 0)),
            pl.BlockSpec((BE, 2), lambda i: (i, 0)),
            _const2((1, D)), _const2((1, D)), _const2((1, D)),
            _const2((D, 5)), _const2((2, 5)), _const2((1, 5)),
        ],
        out_specs=pl.BlockSpec((BE, 5), lambda i: (i, 0)),
        out_shape=jax.ShapeDtypeStruct((E, 5), jnp.float32),
    )(xs, xs, edge_feat,
      p['b1'].reshape(1, D), p['g_pred'].reshape(1, D), p['beta_pred'].reshape(1, D),
      w2[:, :D].T, w2[:, D:].T, p['b2'].reshape(1, 5))


def kernel(h, edge_weight, edge_feat, params, edge_index):
    p = params
    src = edge_index[0]
    dst = edge_index[1]
    pad = EPADX - E
    srcp = jnp.concatenate([src, jnp.zeros((pad,), jnp.int32)])
    dstp = jnp.concatenate([dst, jnp.full((pad,), N, jnp.int32)])
    wp = jnp.concatenate([edge_weight, jnp.zeros((pad,), jnp.float32)])
    wp16 = jnp.broadcast_to(wp[:, None], (EPADX, 16)).reshape(EPADX * 16)
    src3 = srcp[:EPAD].reshape(NSUB, NBATCH, B)
    dst3 = dstp[:EPAD].reshape(NSUB, NBATCH, B)


    hh_s = _run_proj(h, p)
    deg = _sc_deg(dstp)
    for l in range(3):
        agg = _sc_agg(hh_s.reshape(2 * N, HD), srcp, dst3, wp16)
        hh_s = _run_layer(hh_s, agg, deg, p, l)
    a_s, b_s = _run_ab(hh_s, p)
    xs = _sc_edge_gather(a_s.reshape(2 * N, HD), b_s.reshape(2 * N, HD),
                         src3, dst3)
    return _run_edge_mlp(xs, edge_feat, p)
    for l in range(3):
        agg = _sc_agg(hh_s.reshape(2 * N, HD), srcp, dstp, wp)
        hh_s = _run_layer(hh_s, agg, deg, p, l)
    a_s, b_s = _run_ab(hh_s, p)
    xs = _sc_edge_gather(a_s.reshape(2 * N, HD), b_s.reshape(2 * N, HD),
                         srcp, dstp)
    return _run_edge_mlp(xs, edge_feat, p)
